# Initial kernel scaffold; baseline (speedup 1.0000x reference)
import jax
import jax.numpy as jnp
from jax.experimental import pallas as pl


def kernel(x, W0, W1, W2, W3, W4, W5, W6, W7, W8):
    raise NotImplementedError("write your pallas kernel here")


# speedup vs baseline: 3.3343x; 3.3343x over previous
import functools

import jax
import jax.numpy as jnp
import numpy as np
from jax import lax
from jax.experimental import pallas as pl
from jax.experimental.pallas import tpu as pltpu
from jax.experimental.pallas import tpu_sc as plsc

_DIMS = [119, 4, 12, 12, 9, 5, 6, 2, 2]
_OFFS = np.concatenate([[0], np.cumsum(_DIMS)[:-1]]).astype(np.int32)
_TOT = int(sum(_DIMS))
_EMB = 128
_N = 100000
_NC, _NS = 2, 16
_NW = _NC * _NS
_CH = 128
_RPT = 3200
_NPAD = _NW * _RPT


def _sc_body(tbl_hbm, idx_hbm, out_hbm, tbl_v, idx_v, stage_v):
    wid = lax.axis_index("s") * _NC + lax.axis_index("c")
    base = wid * _RPT
    pltpu.sync_copy(tbl_hbm, tbl_v)
    iota = lax.iota(jnp.int32, 16)

    def chunk_body(k, _):
        row0 = base + k * _CH
        pltpu.sync_copy(idx_hbm.at[pl.ds(row0, _CH)], idx_v)

        def row_body(i, _):
            ivec = idx_v[i, :]
            for c in range(_EMB // 16):
                acc = None
                for j in range(9):
                    flat = ivec[j] * _EMB + (c * 16) + iota
                    v = plsc.load_gather(tbl_v, [flat])
                    acc = v if acc is None else acc + v
                stage_v[i, pl.ds(c * 16, 16)] = acc
            return 0

        lax.fori_loop(0, _CH, row_body, 0, unroll=False)
        pltpu.sync_copy(stage_v, out_hbm.at[pl.ds(row0, _CH)])
        return 0

    lax.fori_loop(0, _RPT // _CH, chunk_body, 0, unroll=False)


@functools.partial(jax.jit, static_argnames=())
def kernel(x, W0, W1, W2, W3, W4, W5, W6, W7, W8):
    tbl = jnp.concatenate([W0, W1, W2, W3, W4, W5, W6, W7, W8], axis=0)
    tbl_flat = tbl.reshape(-1)
    idx = x.astype(jnp.int32) + _OFFS[None, :]
    idx_pad = jnp.zeros((_NPAD, 16), jnp.int32).at[:_N, :9].set(idx)

    run = pl.kernel(
        _sc_body,
        out_type=jax.ShapeDtypeStruct((_NPAD, _EMB), jnp.float32),
        mesh=plsc.VectorSubcoreMesh(
            core_axis_name="c", subcore_axis_name="s", num_cores=_NC
        ),
        scratch_types=[
            pltpu.VMEM((_TOT * _EMB,), jnp.float32),
            pltpu.VMEM((_CH, 16), jnp.int32),
            pltpu.VMEM((_CH, _EMB), jnp.float32),
        ],
        compiler_params=pltpu.CompilerParams(needs_layout_passes=False),
    )
    out = run(tbl_flat, idx_pad)
    return out[:_N]
